# Initial kernel scaffold; baseline (speedup 1.0000x reference)
import functools

import jax
import jax.numpy as jnp
from jax import lax
from jax.experimental import pallas as pl
from jax.experimental.pallas import tpu as pltpu
from jax.experimental.pallas import tpu_sc as plsc

N_NODES = 10000
N_EDGES = 320000
D = 128
DE = 16
NEG = 0.01

NC = 2
NS = 16
NW = NC * NS

C = 128
NCHUNKS = N_EDGES // C
BASE_ITERS = NCHUNKS // NW
EXTRA = NCHUNKS % NW

ROWS_PER_TILE = N_NODES // NS


def _leaky(v):
    return jnp.where(v >= 0, v, NEG * v)


def _prep_body(x_ref, wt_ref, b_ref, h_ref):
    v = jnp.dot(x_ref[...], wt_ref[...], preferred_element_type=jnp.float32)
    h_ref[...] = _leaky(v + b_ref[...])


def _prep(x, wt, b):
    rb = 1250
    return pl.pallas_call(
        _prep_body,
        grid=(N_NODES // rb,),
        in_specs=[
            pl.BlockSpec((rb, D), lambda i: (i, 0)),
            pl.BlockSpec((D, D), lambda i: (0, 0)),
            pl.BlockSpec((1, D), lambda i: (0, 0)),
        ],
        out_specs=pl.BlockSpec((rb, D), lambda i: (i, 0)),
        out_shape=jax.ShapeDtypeStruct((N_NODES, D), jnp.float32),
    )(x, wt, b)


def _edge_body(ew_ref, wt_ref, b_ref, e_ref):
    v = jnp.dot(ew_ref[...], wt_ref[...], preferred_element_type=jnp.float32)
    e_ref[...] = v + b_ref[...]


def _edge_lin(ew, wt, b):
    eb = 8000
    return pl.pallas_call(
        _edge_body,
        grid=(N_EDGES // eb,),
        in_specs=[
            pl.BlockSpec((eb, DE), lambda i: (i, 0)),
            pl.BlockSpec((DE, D), lambda i: (0, 0)),
            pl.BlockSpec((1, D), lambda i: (0, 0)),
        ],
        out_specs=pl.BlockSpec((eb, D), lambda i: (i, 0)),
        out_shape=jax.ShapeDtypeStruct((N_EDGES, D), jnp.float32),
    )(ew, wt, b)


def _sc_agg_body(h_hbm, src_hbm, dst_hbm, e_hbm, out_hbm,
                 src_v, dst_v, hm_v, e_v, agg_sh, sem_g, sem_e):
    cid = lax.axis_index("c")
    sid = lax.axis_index("s")
    wid = sid * NC + cid

    zero = jnp.zeros((16,), jnp.float32)

    def zbody(i, _):
        for j in range(D // 16):
            e_v[i, pl.ds(j * 16, 16)] = zero
        return 0

    lax.fori_loop(0, C, zbody, 0)

    r0 = sid * ROWS_PER_TILE
    full = ROWS_PER_TILE // C
    tail = ROWS_PER_TILE - full * C
    for t in range(full):
        pltpu.sync_copy(e_v, agg_sh.at[pl.ds(r0 + t * C, C)])
    pltpu.sync_copy(e_v.at[pl.ds(0, tail)],
                    agg_sh.at[pl.ds(r0 + full * C, tail)])
    plsc.subcore_barrier()

    nit = BASE_ITERS + jnp.where(wid < EXTRA, 1, 0)

    def body(k, _):
        base = (wid + k * NW) * C
        pltpu.sync_copy(src_hbm.at[pl.ds(base, C)], src_v)
        pltpu.sync_copy(dst_hbm.at[pl.ds(base, C)], dst_v)
        cp_g = pltpu.async_copy(h_hbm.at[src_v], hm_v, sem_g)
        cp_e = pltpu.async_copy(e_hbm.at[pl.ds(base, C)], e_v, sem_e)
        cp_g.wait()
        cp_e.wait()

        def cbody(i, _):
            for j in range(D // 16):
                s = pl.ds(j * 16, 16)
                hm_v[i, s] = jnp.maximum(hm_v[i, s] + e_v[i, s], 0.0)
            return 0

        lax.fori_loop(0, C, cbody, 0)
        pltpu.sync_copy(hm_v, agg_sh.at[dst_v], add=True)
        return 0

    lax.fori_loop(0, nit, body, 0)
    plsc.subcore_barrier()

    pltpu.sync_copy(agg_sh.at[pl.ds(r0, ROWS_PER_TILE)],
                    out_hbm.at[cid, pl.ds(r0, ROWS_PER_TILE)])


_sc_agg = pl.kernel(
    _sc_agg_body,
    out_type=jax.ShapeDtypeStruct((NC, N_NODES, D), jnp.float32),
    mesh=plsc.VectorSubcoreMesh(core_axis_name="c", subcore_axis_name="s"),
    scratch_types=[
        pltpu.VMEM((C,), jnp.int32),
        pltpu.VMEM((C,), jnp.int32),
        pltpu.VMEM((C, D), jnp.float32),
        pltpu.VMEM((C, D), jnp.float32),
        pltpu.VMEM_SHARED((N_NODES, D), jnp.float32),
        pltpu.SemaphoreType.DMA,
        pltpu.SemaphoreType.DMA,
    ],
)


def _post_body(a0_ref, a1_ref, h_ref, w1t, b1r, w2t, b2r, wpz, wph, bp, out_ref):
    h = h_ref[...]
    z = a0_ref[...] + a1_ref[...] + h
    z = _leaky(jnp.dot(z, w1t[...], preferred_element_type=jnp.float32) + b1r[...])
    z = jnp.tanh(jnp.dot(z, w2t[...], preferred_element_type=jnp.float32) + b2r[...])
    o = (jnp.dot(z, wpz[...], preferred_element_type=jnp.float32)
         + jnp.dot(h, wph[...], preferred_element_type=jnp.float32) + bp[...])
    out_ref[...] = jnp.tanh(o)


def _post(a0, a1, h, w1t, b1, w2t, b2, wpz, wph, bp):
    rb = 1250
    mat = pl.BlockSpec((rb, D), lambda i: (i, 0))
    wsp = pl.BlockSpec((D, D), lambda i: (0, 0))
    bsp = pl.BlockSpec((1, D), lambda i: (0, 0))
    return pl.pallas_call(
        _post_body,
        grid=(N_NODES // rb,),
        in_specs=[mat, mat, mat, wsp, bsp, wsp, bsp, wsp, wsp, bsp],
        out_specs=mat,
        out_shape=jax.ShapeDtypeStruct((N_NODES, D), jnp.float32),
    )(a0, a1, h, w1t, b1, w2t, b2, wpz, wph, bp)


@jax.jit
def kernel(x, edge_index, edge_weight, W_prep, b_prep, W_e, b_e,
           W1, b1, W2, b2, W_post, b_post):
    h = _prep(x, W_prep.T, b_prep.reshape(1, D))
    e = _edge_lin(edge_weight, W_e.T, b_e.reshape(1, D))
    agg2 = _sc_agg(h, edge_index[0], edge_index[1], e)
    return _post(agg2[0], agg2[1], h,
                 W1.T, b1.reshape(1, D), W2.T, b2.reshape(1, D),
                 W_post[:, :D].T, W_post[:, D:].T, b_post.reshape(1, D))


# speedup vs baseline: 3.0353x; 3.0353x over previous
import functools

import jax
import jax.numpy as jnp
from jax import lax
from jax.experimental import pallas as pl
from jax.experimental.pallas import tpu as pltpu
from jax.experimental.pallas import tpu_sc as plsc

N_NODES = 10000
N_EDGES = 320000
D = 128
DE = 16
NEG = 0.01

NC = 2
NS = 16
NW = NC * NS

C = 128
NCHUNKS = N_EDGES // C
BASE_ITERS = NCHUNKS // NW
EXTRA = NCHUNKS % NW

N_PAD = 10240
ROWS_PER_TILE = N_PAD // NS


def _leaky(v):
    return jnp.where(v >= 0, v, NEG * v)


def _prep_body(x_ref, wt_ref, b_ref, h_ref):
    v = jnp.dot(x_ref[...], wt_ref[...], preferred_element_type=jnp.float32)
    h_ref[...] = _leaky(v + b_ref[...])


def _prep(x, wt, b):
    rb = 2000
    return pl.pallas_call(
        _prep_body,
        grid=(N_NODES // rb,),
        in_specs=[
            pl.BlockSpec((rb, D), lambda i: (i, 0)),
            pl.BlockSpec((D, D), lambda i: (0, 0)),
            pl.BlockSpec((1, D), lambda i: (0, 0)),
        ],
        out_specs=pl.BlockSpec((rb, D), lambda i: (i, 0)),
        out_shape=jax.ShapeDtypeStruct((N_NODES, D), jnp.float32),
    )(x, wt, b)


def _edge_body(ew_ref, wt_ref, b_ref, e_ref):
    v = jnp.dot(ew_ref[...], wt_ref[...], preferred_element_type=jnp.float32)
    e_ref[...] = v + b_ref[...]


def _edge_lin(ew, wt, b):
    eb = 8000
    return pl.pallas_call(
        _edge_body,
        grid=(N_EDGES // eb,),
        in_specs=[
            pl.BlockSpec((eb, DE), lambda i: (i, 0)),
            pl.BlockSpec((DE, D), lambda i: (0, 0)),
            pl.BlockSpec((1, D), lambda i: (0, 0)),
        ],
        out_specs=pl.BlockSpec((eb, D), lambda i: (i, 0)),
        out_shape=jax.ShapeDtypeStruct((N_EDGES, D), jnp.float32),
    )(ew, wt, b)


def _sc_agg_body(h_hbm, src_hbm, dst_hbm, e_hbm, out_hbm,
                 src_v, dst_v, hm_v, e_v, agg_sh, sem_g, sem_e):
    cid = lax.axis_index("c")
    sid = lax.axis_index("s")
    wid = sid * NC + cid

    zero = jnp.zeros((16,), jnp.float32)

    def zbody(i, _):
        for j in range(D // 16):
            e_v[i, pl.ds(j * 16, 16)] = zero
        return 0

    lax.fori_loop(0, C, zbody, 0)

    r0 = sid * ROWS_PER_TILE
    for t in range(ROWS_PER_TILE // C):
        pltpu.sync_copy(e_v, agg_sh.at[pl.ds(r0 + t * C, C)])
    plsc.subcore_barrier()

    nit = BASE_ITERS + jnp.where(wid < EXTRA, 1, 0)

    def body(k, _):
        base = (wid + k * NW) * C
        pltpu.sync_copy(src_hbm.at[pl.ds(base, C)], src_v)
        pltpu.sync_copy(dst_hbm.at[pl.ds(base, C)], dst_v)
        cp_g = pltpu.async_copy(h_hbm.at[src_v], hm_v, sem_g)
        cp_e = pltpu.async_copy(e_hbm.at[pl.ds(base, C)], e_v, sem_e)
        cp_g.wait()
        cp_e.wait()

        def cbody(i, _):
            for j in range(D // 16):
                s = pl.ds(j * 16, 16)
                hm_v[i, s] = jnp.maximum(hm_v[i, s] + e_v[i, s], 0.0)
            return 0

        lax.fori_loop(0, C, cbody, 0)
        pltpu.sync_copy(hm_v, agg_sh.at[dst_v], add=True)
        return 0

    lax.fori_loop(0, nit, body, 0)
    plsc.subcore_barrier()

    pltpu.sync_copy(agg_sh.at[pl.ds(r0, ROWS_PER_TILE)],
                    out_hbm.at[cid, pl.ds(r0, ROWS_PER_TILE)])


_sc_agg = pl.kernel(
    _sc_agg_body,
    out_type=jax.ShapeDtypeStruct((NC, N_PAD, D), jnp.float32),
    mesh=plsc.VectorSubcoreMesh(core_axis_name="c", subcore_axis_name="s"),
    scratch_types=[
        pltpu.VMEM((C,), jnp.int32),
        pltpu.VMEM((C,), jnp.int32),
        pltpu.VMEM((C, D), jnp.float32),
        pltpu.VMEM((C, D), jnp.float32),
        pltpu.VMEM_SHARED((N_PAD, D), jnp.float32),
        pltpu.SemaphoreType.DMA,
        pltpu.SemaphoreType.DMA,
    ],
)


def _post_body(a0_ref, a1_ref, h_ref, w1t, b1r, w2t, b2r, wpz, wph, bp, out_ref):
    h = h_ref[...]
    z = a0_ref[...] + a1_ref[...] + h
    z = _leaky(jnp.dot(z, w1t[...], preferred_element_type=jnp.float32) + b1r[...])
    z = jnp.tanh(jnp.dot(z, w2t[...], preferred_element_type=jnp.float32) + b2r[...])
    o = (jnp.dot(z, wpz[...], preferred_element_type=jnp.float32)
         + jnp.dot(h, wph[...], preferred_element_type=jnp.float32) + bp[...])
    out_ref[...] = jnp.tanh(o)


def _post(a0, a1, h, w1t, b1, w2t, b2, wpz, wph, bp):
    rb = 2000
    mat = pl.BlockSpec((rb, D), lambda i: (i, 0))
    wsp = pl.BlockSpec((D, D), lambda i: (0, 0))
    bsp = pl.BlockSpec((1, D), lambda i: (0, 0))
    return pl.pallas_call(
        _post_body,
        grid=(N_NODES // rb,),
        in_specs=[mat, mat, mat, wsp, bsp, wsp, bsp, wsp, wsp, bsp],
        out_specs=mat,
        out_shape=jax.ShapeDtypeStruct((N_NODES, D), jnp.float32),
    )(a0, a1, h, w1t, b1, w2t, b2, wpz, wph, bp)


@jax.jit
def kernel(x, edge_index, edge_weight, W_prep, b_prep, W_e, b_e,
           W1, b1, W2, b2, W_post, b_post):
    h = _prep(x, W_prep.T, b_prep.reshape(1, D))
    e = _edge_lin(edge_weight, W_e.T, b_e.reshape(1, D))
    agg2 = _sc_agg(h, edge_index[0], edge_index[1], e)
    return _post(agg2[0], agg2[1], h,
                 W1.T, b1.reshape(1, D), W2.T, b2.reshape(1, D),
                 W_post[:, :D].T, W_post[:, D:].T, b_post.reshape(1, D))


# speedup vs baseline: 4.1499x; 1.3672x over previous
import jax
import jax.numpy as jnp
from jax import lax
from jax.experimental import pallas as pl
from jax.experimental.pallas import tpu as pltpu
from jax.experimental.pallas import tpu_sc as plsc

N_NODES = 10000
N_EDGES = 320000
D = 128
DE = 16
NEG = 0.01

NC = 2
NS = 16
NW = NC * NS

EPW = N_EDGES // NW
C = 80
NCH = EPW // C
NPAIR = (NCH - 1) // 2

N_PAD = 10240
ROWS_PER_TILE = N_PAD // NS

def _leaky(v):
    return jnp.where(v >= 0, v, NEG * v)


def _prep_body(x_ref, wt_ref, b_ref, h_ref):
    v = jnp.dot(x_ref[...], wt_ref[...], preferred_element_type=jnp.float32)
    h_ref[...] = _leaky(v + b_ref[...])


def _prep(x, wt, b):
    rb = 2000
    return pl.pallas_call(
        _prep_body,
        grid=(N_NODES // rb,),
        in_specs=[
            pl.BlockSpec((rb, D), lambda i: (i, 0)),
            pl.BlockSpec((D, D), lambda i: (0, 0)),
            pl.BlockSpec((1, D), lambda i: (0, 0)),
        ],
        out_specs=pl.BlockSpec((rb, D), lambda i: (i, 0)),
        out_shape=jax.ShapeDtypeStruct((N_NODES, D), jnp.float32),
    )(x, wt, b)


def _edge_body(ewt_ref, w_ref, b_ref, e_ref):
    v = lax.dot_general(ewt_ref[...], w_ref[...],
                        dimension_numbers=(((0,), (0,)), ((), ())),
                        preferred_element_type=jnp.float32)
    v = v + b_ref[...]
    bits = lax.bitcast_convert_type(v, jnp.uint32)
    rne = bits + jnp.uint32(0x7FFF) + ((bits >> jnp.uint32(16)) & jnp.uint32(1))
    lows = jnp.concatenate(
        [rne[:, 32 * q:32 * q + 16] for q in range(D // 32)], axis=1)
    highs = jnp.concatenate(
        [rne[:, 32 * q + 16:32 * q + 32] for q in range(D // 32)], axis=1)
    e_ref[...] = (lows >> jnp.uint32(16)) | (highs & jnp.uint32(0xFFFF0000))


def _edge_lin(ewt, w, b):
    eb = 16000
    return pl.pallas_call(
        _edge_body,
        grid=(N_EDGES // eb,),
        in_specs=[
            pl.BlockSpec((DE, eb), lambda i: (0, i)),
            pl.BlockSpec((DE, D), lambda i: (0, 0)),
            pl.BlockSpec((1, D), lambda i: (0, 0)),
        ],
        out_specs=pl.BlockSpec((eb, D // 2), lambda i: (i, 0)),
        out_shape=jax.ShapeDtypeStruct((N_EDGES, D // 2), jnp.uint32),
    )(ewt, w, b)


def _sc_agg_body(h_hbm, src_hbm, dst_hbm, e_hbm, out_hbm,
                 sb0, sb1, db0, db1, hg0, hg1, ev0, ev1, agg_sh,
                 si0, si1, sd0, sd1, sg0, sg1, se0, se1):
    cid = lax.axis_index("c")
    sid = lax.axis_index("s")
    wid = sid * NC + cid
    ebase = wid * EPW

    sb = (sb0, sb1)
    db = (db0, db1)
    hg = (hg0, hg1)
    ev = (ev0, ev1)
    si = (si0, si1)
    sd = (sd0, sd1)
    sg = (sg0, sg1)
    se = (se0, se1)

    zero = jnp.zeros((16,), jnp.float32)

    def zbody(i, _):
        for j in range(D // 16):
            hg0[i, pl.ds(j * 16, 16)] = zero
        return 0

    lax.fori_loop(0, C, zbody, 0)
    r0 = sid * ROWS_PER_TILE
    for t in range(ROWS_PER_TILE // C):
        pltpu.sync_copy(hg0, agg_sh.at[pl.ds(r0 + t * C, C)])
    plsc.subcore_barrier()

    def issue_src(b, c):
        pltpu.async_copy(src_hbm.at[pl.ds(ebase + c * C, C)], sb[b], si[b])

    def issue_dst(b, c):
        pltpu.async_copy(dst_hbm.at[pl.ds(ebase + c * C, C)], db[b], sd[b])

    def issue_data(b, c):
        pltpu.async_copy(h_hbm.at[sb[b]], hg[b], sg[b])
        pltpu.async_copy(e_hbm.at[pl.ds(ebase + c * C, C)], ev[b], se[b])

    def wait_src(b):
        pltpu.make_async_copy(src_hbm.at[pl.ds(0, C)], sb[b], si[b]).wait()

    def wait_dst(b):
        pltpu.make_async_copy(dst_hbm.at[pl.ds(0, C)], db[b], sd[b]).wait()

    def wait_data(b):
        pltpu.make_async_copy(h_hbm.at[sb[b]], hg[b], sg[b]).wait()
        pltpu.make_async_copy(e_hbm.at[pl.ds(0, C)], ev[b], se[b]).wait()

    def compute(b):
        def cbody(i, _):
            for q in range(D // 32):
                w = ev[b][i, pl.ds(16 * q, 16)]
                u = lax.bitcast_convert_type(w << jnp.uint32(16), jnp.float32)
                v = lax.bitcast_convert_type(w & jnp.uint32(0xFFFF0000), jnp.float32)
                s0 = pl.ds(32 * q, 16)
                s1 = pl.ds(32 * q + 16, 16)
                hg[b][i, s0] = jnp.maximum(hg[b][i, s0] + u, 0.0)
                hg[b][i, s1] = jnp.maximum(hg[b][i, s1] + v, 0.0)
            return 0
        lax.fori_loop(0, C, cbody, 0)

    issue_src(0, 0)
    issue_dst(0, 0)
    issue_src(1, 1)
    issue_dst(1, 1)
    wait_src(0)
    issue_data(0, 0)

    def body(j, _):
        for b in (0, 1):
            nb = 1 - b
            c = 2 * j + b
            c2 = jnp.minimum(c + 2, NCH - 1)
            wait_src(nb)
            issue_data(nb, c + 1)
            wait_data(b)
            issue_src(b, c2)
            compute(b)
            wait_dst(b)
            pltpu.sync_copy(hg[b], agg_sh.at[db[b]], add=True)
            issue_dst(b, c2)
        return 0

    lax.fori_loop(0, NPAIR, body, 0)

    wait_data(0)
    compute(0)
    wait_dst(0)
    pltpu.sync_copy(hg[0], agg_sh.at[db[0]], add=True)
    wait_src(1)
    wait_dst(1)

    plsc.subcore_barrier()

    pltpu.sync_copy(agg_sh.at[pl.ds(r0, ROWS_PER_TILE)],
                    out_hbm.at[cid, pl.ds(r0, ROWS_PER_TILE)])


_sc_agg = pl.kernel(
    _sc_agg_body,
    out_type=jax.ShapeDtypeStruct((NC, N_PAD, D), jnp.float32),
    mesh=plsc.VectorSubcoreMesh(core_axis_name="c", subcore_axis_name="s"),
    scratch_types=[
        pltpu.VMEM((C,), jnp.int32),
        pltpu.VMEM((C,), jnp.int32),
        pltpu.VMEM((C,), jnp.int32),
        pltpu.VMEM((C,), jnp.int32),
        pltpu.VMEM((C, D), jnp.float32),
        pltpu.VMEM((C, D), jnp.float32),
        pltpu.VMEM((C, D // 2), jnp.uint32),
        pltpu.VMEM((C, D // 2), jnp.uint32),
        pltpu.VMEM_SHARED((N_PAD, D), jnp.float32),
        pltpu.SemaphoreType.DMA,
        pltpu.SemaphoreType.DMA,
        pltpu.SemaphoreType.DMA,
        pltpu.SemaphoreType.DMA,
        pltpu.SemaphoreType.DMA,
        pltpu.SemaphoreType.DMA,
        pltpu.SemaphoreType.DMA,
        pltpu.SemaphoreType.DMA,
    ],
)


def _post_body(agg_ref0, agg_ref1, h_ref, w1t, b1r, w2t, b2r, wpz, wph, bp,
               out_ref):
    h = h_ref[...]
    z = agg_ref0[0] + agg_ref1[0] + h
    z = _leaky(jnp.dot(z, w1t[...], preferred_element_type=jnp.float32) + b1r[...])
    z = jnp.tanh(jnp.dot(z, w2t[...], preferred_element_type=jnp.float32) + b2r[...])
    o = (jnp.dot(z, wpz[...], preferred_element_type=jnp.float32)
         + jnp.dot(h, wph[...], preferred_element_type=jnp.float32) + bp[...])
    out_ref[...] = jnp.tanh(o)


def _post(agg2, h, w1t, b1, w2t, b2, wpz, wph, bp):
    rb = 2000
    mat = pl.BlockSpec((rb, D), lambda i: (i, 0))
    wsp = pl.BlockSpec((D, D), lambda i: (0, 0))
    bsp = pl.BlockSpec((1, D), lambda i: (0, 0))
    a0 = pl.BlockSpec((1, rb, D), lambda i: (0, i, 0))
    a1 = pl.BlockSpec((1, rb, D), lambda i: (1, i, 0))
    return pl.pallas_call(
        _post_body,
        grid=(N_NODES // rb,),
        in_specs=[a0, a1, mat, wsp, bsp, wsp, bsp, wsp, wsp, bsp],
        out_specs=mat,
        out_shape=jax.ShapeDtypeStruct((N_NODES, D), jnp.float32),
    )(agg2, agg2, h, w1t, b1, w2t, b2, wpz, wph, bp)


@jax.jit
def kernel(x, edge_index, edge_weight, W_prep, b_prep, W_e, b_e,
           W1, b1, W2, b2, W_post, b_post):
    h = _prep(x, W_prep.T, b_prep.reshape(1, D))
    e = _edge_lin(edge_weight.T, W_e.T, b_e.reshape(1, D))
    agg2 = _sc_agg(h, edge_index[0], edge_index[1], e)
    return _post(agg2, h,
                 W1.T, b1.reshape(1, D), W2.T, b2.reshape(1, D),
                 W_post[:, :D].T, W_post[:, D:].T, b_post.reshape(1, D))


# speedup vs baseline: 6.2984x; 1.5177x over previous
import jax
import jax.numpy as jnp
from jax import lax
from jax.experimental import pallas as pl
from jax.experimental.pallas import tpu as pltpu
from jax.experimental.pallas import tpu_sc as plsc

N_NODES = 10000
N_EDGES = 320000
D = 128
DE = 16
NEG = 0.01

NC = 2
NS = 16
NW = NC * NS

EPW = N_EDGES // NW
C = 80
NCH = EPW // C
NPAIR = (NCH - 1) // 2

N_PAD = 10240
ROWS_PER_TILE = N_PAD // NS

def _leaky(v):
    return jnp.where(v >= 0, v, NEG * v)


def _prep_body(x_ref, wt_ref, b_ref, h_ref):
    v = jnp.dot(x_ref[...], wt_ref[...], preferred_element_type=jnp.float32)
    h_ref[...] = _leaky(v + b_ref[...])


def _prep(x, wt, b):
    rb = 2000
    return pl.pallas_call(
        _prep_body,
        grid=(N_NODES // rb,),
        in_specs=[
            pl.BlockSpec((rb, D), lambda i: (i, 0)),
            pl.BlockSpec((D, D), lambda i: (0, 0)),
            pl.BlockSpec((1, D), lambda i: (0, 0)),
        ],
        out_specs=pl.BlockSpec((rb, D), lambda i: (i, 0)),
        out_shape=jax.ShapeDtypeStruct((N_NODES, D), jnp.float32),
    )(x, wt, b)


def _edge_body(ewt_ref, w_ref, b_ref, e_ref):
    v = lax.dot_general(ewt_ref[...], w_ref[...],
                        dimension_numbers=(((0,), (0,)), ((), ())),
                        preferred_element_type=jnp.float32)
    v = v + b_ref[...]
    bits = lax.bitcast_convert_type(v, jnp.uint32)
    r = bits + jnp.uint32(0x8000)
    e_ref[...] = (r[:, :D // 2] >> jnp.uint32(16)) | (r[:, D // 2:] & jnp.uint32(0xFFFF0000))


def _edge_lin(ewt, w, b):
    eb = 16000
    return pl.pallas_call(
        _edge_body,
        grid=(N_EDGES // eb,),
        in_specs=[
            pl.BlockSpec((DE, eb), lambda i: (0, i)),
            pl.BlockSpec((DE, D), lambda i: (0, 0)),
            pl.BlockSpec((1, D), lambda i: (0, 0)),
        ],
        out_specs=pl.BlockSpec((eb, D // 2), lambda i: (i, 0)),
        out_shape=jax.ShapeDtypeStruct((N_EDGES, D // 2), jnp.uint32),
    )(ewt, w, b)


def _sc_agg_body(h_hbm, src_hbm, dst_hbm, e_hbm, out_hbm,
                 sb0, sb1, db0, db1, hg0, hg1, ev0, ev1, agg_sh,
                 si0, si1, sd0, sd1, sg0, sg1, se0, se1):
    cid = lax.axis_index("c")
    sid = lax.axis_index("s")
    wid = sid * NC + cid
    ebase = wid * EPW

    sb = (sb0, sb1)
    db = (db0, db1)
    hg = (hg0, hg1)
    ev = (ev0, ev1)
    si = (si0, si1)
    sd = (sd0, sd1)
    sg = (sg0, sg1)
    se = (se0, se1)

    zero = jnp.zeros((16,), jnp.float32)

    def zbody(i, _):
        for j in range(D // 16):
            hg0[i, pl.ds(j * 16, 16)] = zero
        return 0

    lax.fori_loop(0, C, zbody, 0)
    r0 = sid * ROWS_PER_TILE
    for t in range(ROWS_PER_TILE // C):
        pltpu.sync_copy(hg0, agg_sh.at[pl.ds(r0 + t * C, C)])
    plsc.subcore_barrier()

    def issue_src(b, c):
        pltpu.async_copy(src_hbm.at[pl.ds(ebase + c * C, C)], sb[b], si[b])

    def issue_dst(b, c):
        pltpu.async_copy(dst_hbm.at[pl.ds(ebase + c * C, C)], db[b], sd[b])

    def issue_data(b, c):
        pltpu.async_copy(h_hbm.at[sb[b]], hg[b], sg[b])
        pltpu.async_copy(e_hbm.at[pl.ds(ebase + c * C, C)], ev[b], se[b])

    def wait_src(b):
        pltpu.make_async_copy(src_hbm.at[pl.ds(0, C)], sb[b], si[b]).wait()

    def wait_dst(b):
        pltpu.make_async_copy(dst_hbm.at[pl.ds(0, C)], db[b], sd[b]).wait()

    def wait_data(b):
        pltpu.make_async_copy(h_hbm.at[sb[b]], hg[b], sg[b]).wait()
        pltpu.make_async_copy(e_hbm.at[pl.ds(0, C)], ev[b], se[b]).wait()

    def compute(b):
        def cbody(i, _):
            for q in range(D // 32):
                w = ev[b][i, pl.ds(16 * q, 16)]
                u = lax.bitcast_convert_type(w << jnp.uint32(16), jnp.float32)
                v = lax.bitcast_convert_type(w & jnp.uint32(0xFFFF0000), jnp.float32)
                s0 = pl.ds(16 * q, 16)
                s1 = pl.ds(D // 2 + 16 * q, 16)
                hg[b][i, s0] = jnp.maximum(hg[b][i, s0] + u, 0.0)
                hg[b][i, s1] = jnp.maximum(hg[b][i, s1] + v, 0.0)
            return 0
        lax.fori_loop(0, C, cbody, 0)

    issue_src(0, 0)
    issue_dst(0, 0)
    issue_src(1, 1)
    issue_dst(1, 1)
    wait_src(0)
    issue_data(0, 0)

    def body(j, _):
        for b in (0, 1):
            nb = 1 - b
            c = 2 * j + b
            c2 = jnp.minimum(c + 2, NCH - 1)
            wait_src(nb)
            issue_data(nb, c + 1)
            wait_data(b)
            issue_src(b, c2)
            compute(b)
            wait_dst(b)
            pltpu.sync_copy(hg[b], agg_sh.at[db[b]], add=True)
            issue_dst(b, c2)
        return 0

    lax.fori_loop(0, NPAIR, body, 0)

    wait_data(0)
    compute(0)
    wait_dst(0)
    pltpu.sync_copy(hg[0], agg_sh.at[db[0]], add=True)
    wait_src(1)
    wait_dst(1)

    plsc.subcore_barrier()

    pltpu.sync_copy(agg_sh.at[pl.ds(r0, ROWS_PER_TILE)],
                    out_hbm.at[cid, pl.ds(r0, ROWS_PER_TILE)])


_sc_agg = pl.kernel(
    _sc_agg_body,
    out_type=jax.ShapeDtypeStruct((NC, N_PAD, D), jnp.float32),
    mesh=plsc.VectorSubcoreMesh(core_axis_name="c", subcore_axis_name="s"),
    scratch_types=[
        pltpu.VMEM((C,), jnp.int32),
        pltpu.VMEM((C,), jnp.int32),
        pltpu.VMEM((C,), jnp.int32),
        pltpu.VMEM((C,), jnp.int32),
        pltpu.VMEM((C, D), jnp.float32),
        pltpu.VMEM((C, D), jnp.float32),
        pltpu.VMEM((C, D // 2), jnp.uint32),
        pltpu.VMEM((C, D // 2), jnp.uint32),
        pltpu.VMEM_SHARED((N_PAD, D), jnp.float32),
        pltpu.SemaphoreType.DMA,
        pltpu.SemaphoreType.DMA,
        pltpu.SemaphoreType.DMA,
        pltpu.SemaphoreType.DMA,
        pltpu.SemaphoreType.DMA,
        pltpu.SemaphoreType.DMA,
        pltpu.SemaphoreType.DMA,
        pltpu.SemaphoreType.DMA,
    ],
)


def _post_body(agg_ref0, agg_ref1, h_ref, w1t, b1r, w2t, b2r, wpz, wph, bp,
               out_ref):
    h = h_ref[...]
    z = agg_ref0[0] + agg_ref1[0] + h
    z = _leaky(jnp.dot(z, w1t[...], preferred_element_type=jnp.float32) + b1r[...])
    z = jnp.tanh(jnp.dot(z, w2t[...], preferred_element_type=jnp.float32) + b2r[...])
    o = (jnp.dot(z, wpz[...], preferred_element_type=jnp.float32)
         + jnp.dot(h, wph[...], preferred_element_type=jnp.float32) + bp[...])
    out_ref[...] = jnp.tanh(o)


def _post(agg2, h, w1t, b1, w2t, b2, wpz, wph, bp):
    rb = 2000
    mat = pl.BlockSpec((rb, D), lambda i: (i, 0))
    wsp = pl.BlockSpec((D, D), lambda i: (0, 0))
    bsp = pl.BlockSpec((1, D), lambda i: (0, 0))
    a0 = pl.BlockSpec((1, rb, D), lambda i: (0, i, 0))
    a1 = pl.BlockSpec((1, rb, D), lambda i: (1, i, 0))
    return pl.pallas_call(
        _post_body,
        grid=(N_NODES // rb,),
        in_specs=[a0, a1, mat, wsp, bsp, wsp, bsp, wsp, wsp, bsp],
        out_specs=mat,
        out_shape=jax.ShapeDtypeStruct((N_NODES, D), jnp.float32),
    )(agg2, agg2, h, w1t, b1, w2t, b2, wpz, wph, bp)


@jax.jit
def kernel(x, edge_index, edge_weight, W_prep, b_prep, W_e, b_e,
           W1, b1, W2, b2, W_post, b_post):
    h = _prep(x, W_prep.T, b_prep.reshape(1, D))
    e = _edge_lin(edge_weight.T, W_e.T, b_e.reshape(1, D))
    agg2 = _sc_agg(h, edge_index[0], edge_index[1], e)
    return _post(agg2, h,
                 W1.T, b1.reshape(1, D), W2.T, b2.reshape(1, D),
                 W_post[:, :D].T, W_post[:, D:].T, b_post.reshape(1, D))
